# Initial kernel scaffold; baseline (speedup 1.0000x reference)
import jax
import jax.numpy as jnp
from jax.experimental import pallas as pl


def kernel(x_lowf, x_highf, exo_x, edge_index_ll, edge_index_hh, edge_index_hl, edge_index_lh, batch_lowf, batch_highf, params):
    raise NotImplementedError("write your pallas kernel here")


# speedup vs baseline: 40.7713x; 40.7713x over previous
import functools

import jax
import jax.numpy as jnp
from jax import lax
from jax.experimental import pallas as pl
from jax.experimental.pallas import tpu as pltpu
from jax.experimental.pallas import tpu_sc as plsc

N = 50000
E = 800000
B = 64
D_IN = 128
NE = 10
GE = 32

NC = 2
NS = 16
NW = NC * NS
L = 16

EB = 128
NBLK = E // EB
SB = 2
NSB = NBLK // SB
CHUNK = 3128
N_PAD = NS * CHUNK
C1A, C1N, C1B = 3200, 7, 3072

BLK = 1000
NBLOCKS = N // BLK


def _mesh():
    return plsc.VectorSubcoreMesh(core_axis_name="c", subcore_axis_name="s")


def _copy1d(src, dst, s):
    @pl.when(s < C1N)
    def _():
        pltpu.sync_copy(src.at[pl.ds(s * C1A, C1A)],
                        dst.at[pl.ds(s * C1A, C1A)])

    @pl.when(s >= C1N)
    def _():
        off = C1N * C1A + (s - C1N) * C1B
        pltpu.sync_copy(src.at[pl.ds(off, C1B)], dst.at[pl.ds(off, C1B)])


def _sc_degrees(dst_ll, dst_hh, z1):
    @functools.partial(
        pl.kernel,
        out_type=tuple(jax.ShapeDtypeStruct((N_PAD,), jnp.float32)
                       for _ in range(4)),
        mesh=_mesh(),
        compiler_params=pltpu.CompilerParams(use_tc_tiling_on_sc=False),
        scratch_types=[
            pltpu.VMEM((3, SB, EB), jnp.int32),
            pltpu.VMEM((3, SB, EB), jnp.int32),
            pltpu.VMEM((EB,), jnp.float32),
            pltpu.VMEM_SHARED((N_PAD,), jnp.float32),
            pltpu.VMEM_SHARED((N_PAD,), jnp.float32),
            pltpu.SemaphoreType.DMA,
            pltpu.SemaphoreType.DMA,
        ],
    )
    def k(dll_h, dhh_h, z_h, oll0_h, oll1_h, ohh0_h, ohh1_h,
          dll_v, dhh_v, ones_v, hll_sh, hhh_sh, sem_i, sem_s):
        c = lax.axis_index("c")
        s = lax.axis_index("s")
        w = s * NC + c
        for t in range(EB // L):
            ones_v[pl.ds(t * L, L)] = jnp.full((L,), 1.0, jnp.float32)
        _copy1d(z_h, hll_sh, s)
        _copy1d(z_h, hhh_sh, s)
        plsc.subcore_barrier()
        nw = (NSB - 1 - w) // NW + 1

        def fire_idx(i):
            b = pl.multiple_of(SB * (w + i * NW), SB)
            q = lax.rem(i, 3)
            pltpu.async_copy(dll_h.at[pl.ds(b, SB)], dll_v.at[q], sem_i)
            pltpu.async_copy(dhh_h.at[pl.ds(b, SB)], dhh_v.at[q], sem_i)

        def wait_idx():
            pltpu.make_async_copy(dll_h.at[pl.ds(0, SB)], dll_v.at[0],
                                  sem_i).wait()
            pltpu.make_async_copy(dhh_h.at[pl.ds(0, SB)], dhh_v.at[0],
                                  sem_i).wait()

        def fire_scatter(i):
            q = lax.rem(i, 3)
            for j in range(SB):
                pltpu.async_copy(ones_v, hll_sh.at[dll_v.at[q, j]], sem_s,
                                 add=True)
                pltpu.async_copy(ones_v, hhh_sh.at[dhh_v.at[q, j]], sem_s,
                                 add=True)

        def wait_scatter():
            for j in range(SB):
                pltpu.make_async_copy(ones_v, hll_sh.at[dll_v.at[0, j]],
                                      sem_s).wait()
                pltpu.make_async_copy(ones_v, hhh_sh.at[dhh_v.at[0, j]],
                                      sem_s).wait()

        fire_idx(0)
        wait_idx()
        fire_idx(1)

        def body(i, carry):
            @pl.when(i > 0)
            def _():
                wait_scatter()

            fire_scatter(i)

            @pl.when(i + 1 < nw)
            def _():
                wait_idx()

                @pl.when(i + 2 < nw)
                def _():
                    fire_idx(i + 2)

            return carry

        lax.fori_loop(0, nw, body, 0)
        wait_scatter()
        plsc.subcore_barrier()

        @pl.when(c == 0)
        def _():
            _copy1d(hll_sh, oll0_h, s)
            _copy1d(hhh_sh, ohh0_h, s)

        @pl.when(c == 1)
        def _():
            _copy1d(hll_sh, oll1_h, s)
            _copy1d(hhh_sh, ohh1_h, s)

    return k(dst_ll, dst_hh, z1)


def _sc_seg_sum(tab, src_b, dst_b, z2):
    @functools.partial(
        pl.kernel,
        out_type=jax.ShapeDtypeStruct((NC, N_PAD, GE), jnp.float32),
        mesh=_mesh(),
        compiler_params=pltpu.CompilerParams(use_tc_tiling_on_sc=False),
        scratch_types=[
            pltpu.VMEM((3, SB, EB), jnp.int32),
            pltpu.VMEM((4, SB, EB), jnp.int32),
            pltpu.VMEM((3, SB, EB, GE), jnp.float32),
            pltpu.VMEM_SHARED((N_PAD, GE), jnp.float32),
            pltpu.SemaphoreType.DMA,
            pltpu.SemaphoreType.DMA,
            pltpu.SemaphoreType.DMA,
            pltpu.SemaphoreType.DMA,
        ],
    )
    def k(tab_h, src_h, dst_h, z_h, out_h, src_v, dst_v, rows_v, acc_sh,
          sem_i, sem_g, sem_g2, sem_s):
        c = lax.axis_index("c")
        s = lax.axis_index("s")
        w = s * NC + c
        pltpu.sync_copy(z_h.at[pl.ds(s * CHUNK, CHUNK)],
                        acc_sh.at[pl.ds(s * CHUNK, CHUNK)])
        plsc.subcore_barrier()
        nw = (NSB - 1 - w) // NW + 1

        def fire_idx(i):
            b = pl.multiple_of(SB * (w + i * NW), SB)
            pltpu.async_copy(src_h.at[pl.ds(b, SB)],
                             src_v.at[lax.rem(i, 3)], sem_i)
            pltpu.async_copy(dst_h.at[pl.ds(b, SB)],
                             dst_v.at[lax.rem(i, 4)], sem_i)

        def wait_idx():
            pltpu.make_async_copy(src_h.at[pl.ds(0, SB)], src_v.at[0],
                                  sem_i).wait()
            pltpu.make_async_copy(dst_h.at[pl.ds(0, SB)], dst_v.at[0],
                                  sem_i).wait()

        def fire_gather(i):
            q3 = lax.rem(i, 3)

            @pl.when(lax.rem(i, 2) == 0)
            def _():
                for j in range(SB):
                    pltpu.async_copy(tab_h.at[src_v.at[q3, j]],
                                     rows_v.at[q3, j], sem_g)

            @pl.when(lax.rem(i, 2) == 1)
            def _():
                for j in range(SB):
                    pltpu.async_copy(tab_h.at[src_v.at[q3, j]],
                                     rows_v.at[q3, j], sem_g2)

        def wait_gather(i):
            @pl.when(lax.rem(i, 2) == 0)
            def _():
                for j in range(SB):
                    pltpu.make_async_copy(tab_h.at[src_v.at[0, j]],
                                          rows_v.at[0, j], sem_g).wait()

            @pl.when(lax.rem(i, 2) == 1)
            def _():
                for j in range(SB):
                    pltpu.make_async_copy(tab_h.at[src_v.at[0, j]],
                                          rows_v.at[0, j], sem_g2).wait()

        def fire_scatter(i):
            q4 = lax.rem(i, 4)
            q3 = lax.rem(i, 3)
            for j in range(SB):
                pltpu.async_copy(rows_v.at[q3, j],
                                 acc_sh.at[dst_v.at[q4, j]], sem_s, add=True)

        def wait_scatter():
            for j in range(SB):
                pltpu.make_async_copy(rows_v.at[0, j],
                                      acc_sh.at[dst_v.at[0, j]],
                                      sem_s).wait()

        fire_idx(0)
        wait_idx()
        fire_gather(0)
        fire_idx(1)
        wait_idx()
        fire_gather(1)
        fire_idx(2)

        def body(i, carry):
            wait_gather(i)

            @pl.when(i > 0)
            def _():
                wait_scatter()

            fire_scatter(i)

            @pl.when(i + 2 < nw)
            def _():
                wait_idx()
                fire_gather(i + 2)

                @pl.when(i + 3 < nw)
                def _():
                    fire_idx(i + 3)

            return carry

        lax.fori_loop(0, nw, body, 0)
        wait_scatter()
        plsc.subcore_barrier()
        pltpu.sync_copy(acc_sh.at[pl.ds(s * CHUNK, CHUNK)],
                        out_h.at[c, pl.ds(s * CHUNK, CHUNK)])

    return k(tab, src_b, dst_b, z2)


def _sc_gat(hs_tab, hd_tab, a, src_b, dst_b, z2, z1):
    @functools.partial(
        pl.kernel,
        out_type=(jax.ShapeDtypeStruct((NC, N_PAD, GE), jnp.float32),
                  jax.ShapeDtypeStruct((N_PAD,), jnp.float32),
                  jax.ShapeDtypeStruct((N_PAD,), jnp.float32)),
        mesh=_mesh(),
        compiler_params=pltpu.CompilerParams(use_tc_tiling_on_sc=False),
        scratch_types=[
            pltpu.VMEM((2, SB, EB), jnp.int32),
            pltpu.VMEM((3, SB, EB), jnp.int32),
            pltpu.VMEM((2 * SB * EB, GE), jnp.float32),
            pltpu.VMEM((SB * EB, GE), jnp.float32),
            pltpu.VMEM((2 * SB * EB,), jnp.float32),
            pltpu.VMEM((GE,), jnp.float32),
            pltpu.VMEM_SHARED((N_PAD, GE), jnp.float32),
            pltpu.VMEM_SHARED((N_PAD,), jnp.float32),
            pltpu.SemaphoreType.DMA,
            pltpu.SemaphoreType.DMA,
            pltpu.SemaphoreType.DMA,
        ],
    )
    def k(hs_h, hd_h, a_h, src_h, dst_h, z2_h, z1_h, acc_o, s0_o, s1_o,
          src_v, dst_v, hs_v, hd_v, w_v, a_v, acc_sh, s_sh,
          sem_i, sem_g, sem_s):
        c = lax.axis_index("c")
        s = lax.axis_index("s")
        w = s * NC + c
        pltpu.sync_copy(a_h, a_v)
        pltpu.sync_copy(z2_h.at[pl.ds(s * CHUNK, CHUNK)],
                        acc_sh.at[pl.ds(s * CHUNK, CHUNK)])
        _copy1d(z1_h, s_sh, s)
        plsc.subcore_barrier()
        lane = lax.iota(jnp.int32, L)
        a_lo = a_v[pl.ds(0, L)]
        a_hi = a_v[pl.ds(L, L)]
        nw = (NSB - 1 - w) // NW + 1

        def fire_idx(i):
            b = pl.multiple_of(SB * (w + i * NW), SB)
            q3 = lax.rem(i, 3)
            q2 = lax.rem(i, 2)
            pltpu.async_copy(src_h.at[pl.ds(b, SB)], src_v.at[q2], sem_i)
            pltpu.async_copy(dst_h.at[pl.ds(b, SB)], dst_v.at[q3], sem_i)

        def wait_idx():
            pltpu.make_async_copy(src_h.at[pl.ds(0, SB)], src_v.at[0],
                                  sem_i).wait()
            pltpu.make_async_copy(dst_h.at[pl.ds(0, SB)], dst_v.at[0],
                                  sem_i).wait()

        def fire_gather(i):
            q3 = lax.rem(i, 3)
            q2 = lax.rem(i, 2)
            for j in range(SB):
                pltpu.async_copy(hs_h.at[src_v.at[q2, j]],
                                 hs_v.at[pl.ds((q2 * SB + j) * EB, EB)],
                                 sem_g)
                pltpu.async_copy(hd_h.at[dst_v.at[q3, j]],
                                 hd_v.at[pl.ds(j * EB, EB)], sem_g)

        def wait_gather():
            for j in range(SB):
                pltpu.make_async_copy(hs_h.at[src_v.at[0, j]],
                                      hs_v.at[pl.ds(j * EB, EB)],
                                      sem_g).wait()
                pltpu.make_async_copy(hd_h.at[dst_v.at[0, j]],
                                      hd_v.at[pl.ds(j * EB, EB)],
                                      sem_g).wait()

        def fire_scatter(i):
            q3 = lax.rem(i, 3)
            q2 = lax.rem(i, 2)
            for j in range(SB):
                pltpu.async_copy(hs_v.at[pl.ds((q2 * SB + j) * EB, EB)],
                                 acc_sh.at[dst_v.at[q3, j]], sem_s, add=True)
                pltpu.async_copy(w_v.at[pl.ds((q2 * SB + j) * EB, EB)],
                                 s_sh.at[dst_v.at[q3, j]], sem_s, add=True)

        def wait_scatter():
            for j in range(SB):
                pltpu.make_async_copy(hs_v.at[pl.ds(j * EB, EB)],
                                      acc_sh.at[dst_v.at[0, j]],
                                      sem_s).wait()
                pltpu.make_async_copy(w_v.at[pl.ds(j * EB, EB)],
                                      s_sh.at[dst_v.at[0, j]],
                                      sem_s).wait()

        fire_idx(0)
        wait_idx()
        fire_gather(0)
        fire_idx(1)

        def body(i, carry):
            q2 = lax.rem(i, 2)
            wait_gather()

            base = q2 * (SB * EB)

            @plsc.parallel_loop(0, SB * EB, unroll=8)
            def _edge(t):
                ts = base + t
                hs_lo = hs_v[ts, pl.ds(0, L)]
                hs_hi = hs_v[ts, pl.ds(L, L)]
                hd_lo = hd_v[t, pl.ds(0, L)]
                hd_hi = hd_v[t, pl.ds(L, L)]
                t0 = hs_lo + hd_lo
                t1 = hs_hi + hd_hi
                u = (jnp.maximum(t0, t0 * 0.2) * a_lo
                     + jnp.maximum(t1, t1 * 0.2) * a_hi)
                for sh in (8, 4, 2, 1):
                    u = u + u.at[lane ^ sh].get(mode='promise_in_bounds')
                hd_v[t, pl.ds(0, L)] = u

            @plsc.parallel_loop(0, SB * EB // L, unroll=2)
            def _compact(g):
                r0 = g * L
                e16 = jnp.full((L,), 0.0, jnp.float32)
                for j in range(L):
                    erow = hd_v[r0 + j, pl.ds(0, L)]
                    e16 = jnp.where(lane == j, erow, e16)
                w_v[pl.ds(base + g * L, L)] = jnp.exp(e16)

            @plsc.parallel_loop(0, SB * EB, unroll=8)
            def _scale(t):
                ts = base + t
                g = t // L
                w16 = w_v[pl.ds(base + g * L, L)]
                wv = w16.at[jnp.broadcast_to(lax.rem(t, L), (L,))].get(
                    mode='promise_in_bounds')
                hs_v[ts, pl.ds(0, L)] = hs_v[ts, pl.ds(0, L)] * wv
                hs_v[ts, pl.ds(L, L)] = hs_v[ts, pl.ds(L, L)] * wv

            @pl.when(i > 0)
            def _():
                wait_scatter()

            fire_scatter(i)

            @pl.when(i + 1 < nw)
            def _():
                wait_idx()
                fire_gather(i + 1)

                @pl.when(i + 2 < nw)
                def _():
                    fire_idx(i + 2)

            return carry

        lax.fori_loop(0, nw, body, 0)
        wait_scatter()
        plsc.subcore_barrier()
        pltpu.sync_copy(acc_sh.at[pl.ds(s * CHUNK, CHUNK)],
                        acc_o.at[c, pl.ds(s * CHUNK, CHUNK)])

        @pl.when(c == 0)
        def _():
            _copy1d(s_sh, s0_o, s)

        @pl.when(c == 1)
        def _():
            _copy1d(s_sh, s1_o, s)

    return k(hs_tab, hd_tab, a, src_b, dst_b, z2, z1)


def _enc_tables_tc(x_lowf, x_highf, exo_x, b_l2, b_h2, degll, deghh, p):
    f32 = jnp.float32

    def body(xl_ref, xh_ref, exo_ref, bl_ref, bh_ref, dll_ref, dhh_ref,
             We1, be1, We2, be2, Wl, bl, Wh, bh,
             Wll, Wdhl, Wslh, Whh, Wshl, Wdlh,
             tabll_o, hdhl_o, hslh_o, tabhh_o, hshl_o, hdlh_o,
             dinvl_o, dinvh_o):
        exo_hid = jax.nn.relu(
            jax.nn.relu(exo_ref[...] @ We1[...] + be1[...]) @ We2[...]
            + be2[...])
        ebl = exo_hid @ Wl[...][D_IN:, :]
        ebh = exo_hid @ Wh[...][D_IN:, :]
        iot = lax.broadcasted_iota(jnp.int32, (1, B), 1)
        ohl = (bl_ref[...] == iot).astype(f32)
        ohh = (bh_ref[...] == iot).astype(f32)
        xl = jax.nn.relu(xl_ref[...] @ Wl[...][:D_IN, :] + ohl @ ebl + bl[...])
        xh = jax.nn.sigmoid(xh_ref[...] @ Wh[...][:D_IN, :] + ohh @ ebh
                            + bh[...])
        dinvl = lax.rsqrt(1.0 + dll_ref[0] + dll_ref[1])
        dinvh = lax.rsqrt(1.0 + dhh_ref[0] + dhh_ref[1])
        tabll_o[...] = (xl @ Wll[...]) * dinvl
        hdhl_o[...] = xl @ Wdhl[...]
        hslh_o[...] = xl @ Wslh[...]
        tabhh_o[...] = (xh @ Whh[...]) * dinvh
        hshl_o[...] = xh @ Wshl[...]
        hdlh_o[...] = xh @ Wdlh[...]
        dinvl_o[...] = dinvl
        dinvh_o[...] = dinvh

    full = lambda shape: pl.BlockSpec(shape, lambda i: tuple(0 for _ in shape))
    row = lambda shape: pl.BlockSpec(shape, lambda i: (i,) + tuple(
        0 for _ in shape[1:]))
    out32 = jax.ShapeDtypeStruct((N, GE), f32)
    out1 = jax.ShapeDtypeStruct((N, 1), f32)
    return pl.pallas_call(
        body,
        grid=(NBLOCKS,),
        in_specs=[
            row((BLK, D_IN)), row((BLK, D_IN)), full((B, 1)),
            row((BLK, 1)), row((BLK, 1)),
            pl.BlockSpec((NC, BLK, 1), lambda i: (0, i, 0)),
            pl.BlockSpec((NC, BLK, 1), lambda i: (0, i, 0)),
            full((1, NE)), full((1, NE)), full((NE, NE)), full((1, NE)),
            full((D_IN + NE, NE)), full((1, NE)),
            full((D_IN + NE, NE)), full((1, NE)),
            full((NE, GE)), full((NE, GE)), full((NE, GE)),
            full((NE, GE)), full((NE, GE)), full((NE, GE)),
        ],
        out_specs=[row((BLK, GE))] * 6 + [row((BLK, 1))] * 2,
        out_shape=[out32] * 6 + [out1] * 2,
    )(x_lowf, x_highf, exo_x, b_l2, b_h2, degll, deghh,
      p['We1'], p['be1'].reshape(1, -1), p['We2'], p['be2'].reshape(1, -1),
      p['Wl_enc'], p['bl_enc'].reshape(1, -1),
      p['Wh_enc'], p['bh_enc'].reshape(1, -1),
      p['gcn_ll_W0'], p['gat_hl_Wd0'], p['gat_lh_Ws0'],
      p['gcn_hh_W0'], p['gat_hl_Ws0'], p['gat_lh_Wd0'])


def _combine_tc(Sll, tabll, dinvl, Ahl, shl, Shh, tabhh, dinvh, Alh, slh,
                bgl, bal, bgh, bah):
    f32 = jnp.float32

    def body(Sll_ref, tabll_ref, dinvl_ref, Ahl_ref, shl_ref,
             Shh_ref, tabhh_ref, dinvh_ref, Alh_ref, slh_ref,
             bgl_ref, bal_ref, bgh_ref, bah_ref,
             lnew_o, hnew_o, stats_o):
        lnew = (dinvl_ref[...] * (Sll_ref[0] + Sll_ref[1] + tabll_ref[...])
                + bgl_ref[...]
                + (Ahl_ref[0] + Ahl_ref[1])
                / (shl_ref[0] + shl_ref[1] + 1e-16) + bal_ref[...])
        hnew = (dinvh_ref[...] * (Shh_ref[0] + Shh_ref[1] + tabhh_ref[...])
                + bgh_ref[...]
                + (Alh_ref[0] + Alh_ref[1])
                / (slh_ref[0] + slh_ref[1] + 1e-16) + bah_ref[...])
        lnew_o[...] = lnew
        hnew_o[...] = hnew
        ps = jnp.concatenate(
            [jnp.sum(lnew, 0, keepdims=True),
             jnp.sum(lnew * lnew, 0, keepdims=True),
             jnp.sum(hnew, 0, keepdims=True),
             jnp.sum(hnew * hnew, 0, keepdims=True),
             jnp.zeros((4, GE), f32)], 0)

        @pl.when(pl.program_id(0) == 0)
        def _():
            stats_o[...] = ps

        @pl.when(pl.program_id(0) > 0)
        def _():
            stats_o[...] += ps

    full = lambda shape: pl.BlockSpec(shape, lambda i: tuple(0 for _ in shape))
    row = lambda shape: pl.BlockSpec(shape, lambda i: (i,) + tuple(
        0 for _ in shape[1:]))
    part3 = pl.BlockSpec((NC, BLK, GE), lambda i: (0, i, 0))
    part1 = pl.BlockSpec((NC, BLK, 1), lambda i: (0, i, 0))
    return pl.pallas_call(
        body,
        grid=(NBLOCKS,),
        in_specs=[part3, row((BLK, GE)), row((BLK, 1)), part3, part1,
                  part3, row((BLK, GE)), row((BLK, 1)), part3, part1,
                  full((1, GE)), full((1, GE)), full((1, GE)), full((1, GE))],
        out_specs=[row((BLK, GE)), row((BLK, GE)), full((8, GE))],
        out_shape=[jax.ShapeDtypeStruct((N, GE), f32),
                   jax.ShapeDtypeStruct((N, GE), f32),
                   jax.ShapeDtypeStruct((8, GE), f32)],
    )(Sll, tabll, dinvl, Ahl, shl, Shh, tabhh, dinvh, Alh, slh,
      bgl, bal, bgh, bah)


def _bn_tables_tc(lnew, hnew, stats, gl, bl, gh, bh, dinvl, dinvh,
                  Wll, Wdhl, Wslh, Whh, Wshl, Wdlh):
    f32 = jnp.float32
    inv_n = 1.0 / N

    def body(lnew_ref, hnew_ref, stats_ref, gl_ref, bl_ref, gh_ref, bh_ref,
             dinvl_ref, dinvh_ref,
             Wll_ref, Wdhl_ref, Wslh_ref, Whh_ref, Wshl_ref, Wdlh_ref,
             tabll_o, hdhl_o, hslh_o, tabhh_o, hshl_o, hdlh_o):
        mul = stats_ref[0:1] * inv_n
        varl = stats_ref[1:2] * inv_n - mul * mul
        muh = stats_ref[2:3] * inv_n
        varh = stats_ref[3:4] * inv_n - muh * muh
        xl = jax.nn.relu((lnew_ref[...] - mul) * lax.rsqrt(varl + 1e-5)
                         * gl_ref[...] + bl_ref[...])
        xh = jax.nn.relu((hnew_ref[...] - muh) * lax.rsqrt(varh + 1e-5)
                         * gh_ref[...] + bh_ref[...])
        tabll_o[...] = (xl @ Wll_ref[...]) * dinvl_ref[...]
        hdhl_o[...] = xl @ Wdhl_ref[...]
        hslh_o[...] = xl @ Wslh_ref[...]
        tabhh_o[...] = (xh @ Whh_ref[...]) * dinvh_ref[...]
        hshl_o[...] = xh @ Wshl_ref[...]
        hdlh_o[...] = xh @ Wdlh_ref[...]

    full = lambda shape: pl.BlockSpec(shape, lambda i: tuple(0 for _ in shape))
    row = lambda shape: pl.BlockSpec(shape, lambda i: (i,) + tuple(
        0 for _ in shape[1:]))
    return pl.pallas_call(
        body,
        grid=(NBLOCKS,),
        in_specs=[row((BLK, GE)), row((BLK, GE)), full((8, GE)),
                  full((1, GE)), full((1, GE)), full((1, GE)), full((1, GE)),
                  row((BLK, 1)), row((BLK, 1))] + [full((GE, GE))] * 6,
        out_specs=[row((BLK, GE))] * 6,
        out_shape=[jax.ShapeDtypeStruct((N, GE), f32)] * 6,
    )(lnew, hnew, stats, gl, bl, gh, bh, dinvl, dinvh,
      Wll, Wdhl, Wslh, Whh, Wshl, Wdlh)


def _bn_pool_tc(lnew, hnew, stats, gl, bl, gh, bh, b_l2, b_h2):
    f32 = jnp.float32
    inv_n = 1.0 / N

    def body(lnew_ref, hnew_ref, stats_ref, gl_ref, bl_ref, gh_ref, bh_ref,
             bl2_ref, bh2_ref, pooled_o):
        mul = stats_ref[0:1] * inv_n
        varl = stats_ref[1:2] * inv_n - mul * mul
        muh = stats_ref[2:3] * inv_n
        varh = stats_ref[3:4] * inv_n - muh * muh
        xl = jax.nn.relu((lnew_ref[...] - mul) * lax.rsqrt(varl + 1e-5)
                         * gl_ref[...] + bl_ref[...])
        xh = jax.nn.relu((hnew_ref[...] - muh) * lax.rsqrt(varh + 1e-5)
                         * gh_ref[...] + bh_ref[...])
        iot = lax.broadcasted_iota(jnp.int32, (1, B), 1)
        ohl = (bl2_ref[...] == iot).astype(f32)
        ohh = (bh2_ref[...] == iot).astype(f32)
        dn = (((0,), (0,)), ((), ()))
        pool_l = lax.dot_general(ohl, xl, dn)
        pool_h = lax.dot_general(ohh, xh, dn)
        cnt_l = jnp.broadcast_to(jnp.sum(ohl, 0)[:, None], (B, GE))
        cnt_h = jnp.broadcast_to(jnp.sum(ohh, 0)[:, None], (B, GE))
        ps = jnp.concatenate([pool_l, pool_h, cnt_l, cnt_h], 1)

        @pl.when(pl.program_id(0) == 0)
        def _():
            pooled_o[...] = ps

        @pl.when(pl.program_id(0) > 0)
        def _():
            pooled_o[...] += ps

    full = lambda shape: pl.BlockSpec(shape, lambda i: tuple(0 for _ in shape))
    row = lambda shape: pl.BlockSpec(shape, lambda i: (i,) + tuple(
        0 for _ in shape[1:]))
    return pl.pallas_call(
        body,
        grid=(NBLOCKS,),
        in_specs=[row((BLK, GE)), row((BLK, GE)), full((8, GE)),
                  full((1, GE)), full((1, GE)), full((1, GE)), full((1, GE)),
                  row((BLK, 1)), row((BLK, 1))],
        out_specs=[full((B, 4 * GE))],
        out_shape=[jax.ShapeDtypeStruct((B, 4 * GE), f32)],
    )(lnew, hnew, stats, gl, bl, gh, bh, b_l2, b_h2)[0]


def _head_tc(pooled, Wh1, bh1, Wh2, bh2):
    def body(pooled_ref, Wh1_ref, bh1_ref, Wh2_ref, bh2_ref, out_o):
        pp = pooled_ref[...]
        g = jnp.concatenate(
            [pp[:, 0:GE] / jnp.maximum(pp[:, 2 * GE:3 * GE], 1.0),
             pp[:, GE:2 * GE] / jnp.maximum(pp[:, 3 * GE:4 * GE], 1.0)], 1)
        out_o[...] = (jax.nn.relu(g @ Wh1_ref[...] + bh1_ref[...])
                      @ Wh2_ref[...] + bh2_ref[...])

    full = lambda shape: pl.BlockSpec(shape, lambda i: tuple(0 for _ in shape))
    return pl.pallas_call(
        body,
        grid=(1,),
        in_specs=[full((B, 4 * GE)), full((2 * GE, GE)), full((1, GE)),
                  full((GE, 1)), full((1, 1))],
        out_specs=[full((B, 1))],
        out_shape=[jax.ShapeDtypeStruct((B, 1), jnp.float32)],
    )(pooled, Wh1, bh1, Wh2, bh2)[0]


def kernel(x_lowf, x_highf, exo_x, edge_index_ll, edge_index_hh,
           edge_index_hl, edge_index_lh, batch_lowf, batch_highf, params):
    p = params
    f32 = jnp.float32
    src_ll = edge_index_ll[0].reshape(NBLK, EB)
    dst_ll = edge_index_ll[1].reshape(NBLK, EB)
    src_hh = edge_index_hh[0].reshape(NBLK, EB)
    dst_hh = edge_index_hh[1].reshape(NBLK, EB)
    src_hl = edge_index_hl[0].reshape(NBLK, EB)
    dst_hl = edge_index_hl[1].reshape(NBLK, EB)
    src_lh = edge_index_lh[0].reshape(NBLK, EB)
    dst_lh = edge_index_lh[1].reshape(NBLK, EB)
    z1 = jnp.zeros((N_PAD,), f32)
    z2 = jnp.zeros((N_PAD, GE), f32)
    b_l2 = batch_lowf.reshape(N, 1)
    b_h2 = batch_highf.reshape(N, 1)
    r = lambda v: v.reshape(1, -1)

    dll0, dll1, dhh0, dhh1 = _sc_degrees(dst_ll, dst_hh, z1)
    degll_p = jnp.stack([dll0, dll1])
    deghh_p = jnp.stack([dhh0, dhh1])
    tabll, hdhl, hslh, tabhh, hshl, hdlh, dinvl, dinvh = _enc_tables_tc(
        x_lowf, x_highf, exo_x, b_l2, b_h2,
        degll_p[:, :N, None], deghh_p[:, :N, None], p)

    pooled = None
    for l in range(2):
        Sll = _sc_seg_sum(tabll, src_ll, dst_ll, z2)
        Ahl, shl0, shl1 = _sc_gat(hshl, hdhl, p['gat_hl_a%d' % l],
                                  src_hl, dst_hl, z2, z1)
        Shh = _sc_seg_sum(tabhh, src_hh, dst_hh, z2)
        Alh, slh0, slh1 = _sc_gat(hslh, hdlh, p['gat_lh_a%d' % l],
                                  src_lh, dst_lh, z2, z1)
        shl = jnp.stack([shl0, shl1])
        slh = jnp.stack([slh0, slh1])
        lnew, hnew, stats = _combine_tc(
            Sll[:, :N], tabll, dinvl, Ahl[:, :N], shl[:, :N, None],
            Shh[:, :N], tabhh, dinvh, Alh[:, :N], slh[:, :N, None],
            r(p['gcn_ll_b%d' % l]), r(p['gat_hl_b%d' % l]),
            r(p['gcn_hh_b%d' % l]), r(p['gat_lh_b%d' % l]))
        if l == 0:
            tabll, hdhl, hslh, tabhh, hshl, hdlh = _bn_tables_tc(
                lnew, hnew, stats,
                r(p['bn_l_g0']), r(p['bn_l_b0']),
                r(p['bn_h_g0']), r(p['bn_h_b0']),
                dinvl, dinvh,
                p['gcn_ll_W1'], p['gat_hl_Wd1'], p['gat_lh_Ws1'],
                p['gcn_hh_W1'], p['gat_hl_Ws1'], p['gat_lh_Wd1'])
        else:
            pooled = _bn_pool_tc(
                lnew, hnew, stats,
                r(p['bn_l_g1']), r(p['bn_l_b1']),
                r(p['bn_h_g1']), r(p['bn_h_b1']), b_l2, b_h2)

    return _head_tc(pooled, p['Wh1'], r(p['bh1']), p['Wh2'], r(p['bh2']))
